# Initial kernel scaffold; baseline (speedup 1.0000x reference)
import jax
import jax.numpy as jnp
from jax.experimental import pallas as pl


def kernel(pred_y, true_y):
    raise NotImplementedError("write your pallas kernel here")


# speedup vs baseline: 1.1738x; 1.1738x over previous
import jax
import jax.numpy as jnp
from jax.experimental import pallas as pl

_JB = 512
_EPS = 1e-8


def _row_sign_sum(x, n):
    xc = x[:, None]
    acc = jnp.zeros((n, 1), jnp.float32)
    ones = jnp.ones((_JB, 1), jnp.float32)
    for jb in range(n // _JB):
        xb = jax.lax.slice(x, (jb * _JB,), ((jb + 1) * _JB,))[None, :]
        s = jnp.sign(xc - xb)
        acc = acc + jax.lax.dot(s, ones)
    return acc


def _body(x_ref, y_ref, o_ref):
    n = x_ref.shape[-1]
    x = x_ref[0, 0, :]
    y = y_ref[0, 0, :]
    sx = _row_sign_sum(x, n)
    sy = _row_sign_sum(y, n)
    num = jnp.sum(sx * sy) * 0.25
    sxx = jnp.sum(sx * sx) * 0.25
    syy = jnp.sum(sy * sy) * 0.25
    corr = num / jnp.sqrt(sxx * syy + _EPS)
    o_ref[...] = corr[None, None, None]


def kernel(pred_y, true_y):
    b, n = pred_y.shape
    p3 = pred_y.reshape(b, 1, n)
    t3 = true_y.reshape(b, 1, n)
    out = pl.pallas_call(
        _body,
        grid=(b,),
        in_specs=[
            pl.BlockSpec((1, 1, n), lambda r: (r, 0, 0)),
            pl.BlockSpec((1, 1, n), lambda r: (r, 0, 0)),
        ],
        out_specs=pl.BlockSpec((1, 1, 1), lambda r: (r, 0, 0)),
        out_shape=jax.ShapeDtypeStruct((b, 1, 1), jnp.float32),
    )(p3, t3)
    return out[:, 0, 0]


# speedup vs baseline: 1.5967x; 1.3603x over previous
import jax
import jax.numpy as jnp
from jax.experimental import pallas as pl

_BLK = 512
_EPS = 1e-8


def _row_sign_sum(x, n):
    nb = n // _BLK
    xc = x[:, None]
    ones_c = jnp.ones((_BLK, 1), jnp.float32)
    ones_r = jnp.ones((1, _BLK), jnp.float32)
    scol = [jnp.zeros((_BLK, 1), jnp.float32) for _ in range(nb)]
    srow = [jnp.zeros((1, _BLK), jnp.float32) for _ in range(nb)]
    for ib in range(nb):
        xi = jax.lax.slice(xc, (ib * _BLK, 0), ((ib + 1) * _BLK, 1))
        for jb in range(ib, nb):
            xj = jax.lax.slice(x, (jb * _BLK,), ((jb + 1) * _BLK,))[None, :]
            s = jnp.sign(xi - xj)
            scol[ib] = scol[ib] + jax.lax.dot(s, ones_c)
            if jb != ib:
                srow[jb] = srow[jb] - jax.lax.dot(ones_r, s)
    parts = [scol[b][:, 0] + srow[b][0, :] for b in range(nb)]
    return jnp.concatenate(parts)


def _body(x_ref, y_ref, o_ref):
    n = x_ref.shape[-1]
    x = x_ref[0, 0, :]
    y = y_ref[0, 0, :]
    sx = _row_sign_sum(x, n)
    sy = _row_sign_sum(y, n)
    num = jnp.sum(sx * sy) * 0.25
    sxx = jnp.sum(sx * sx) * 0.25
    syy = jnp.sum(sy * sy) * 0.25
    corr = num / jnp.sqrt(sxx * syy + _EPS)
    o_ref[...] = corr[None, None, None]


def kernel(pred_y, true_y):
    b, n = pred_y.shape
    p3 = pred_y.reshape(b, 1, n)
    t3 = true_y.reshape(b, 1, n)
    out = pl.pallas_call(
        _body,
        grid=(b,),
        in_specs=[
            pl.BlockSpec((1, 1, n), lambda r: (r, 0, 0)),
            pl.BlockSpec((1, 1, n), lambda r: (r, 0, 0)),
        ],
        out_specs=pl.BlockSpec((1, 1, 1), lambda r: (r, 0, 0)),
        out_shape=jax.ShapeDtypeStruct((b, 1, 1), jnp.float32),
    )(p3, t3)
    return out[:, 0, 0]


# speedup vs baseline: 9.7546x; 6.1092x over previous
import jax
import jax.numpy as jnp
from jax import lax
from jax.experimental import pallas as pl
from jax.experimental.pallas import tpu as pltpu
from jax.experimental.pallas import tpu_sc as plsc

_N = 4096
_NV = _N // 16
_ROWS = 256
_EPS = 1e-8
_BIG = _N


def _iota16():
    return lax.iota(jnp.int32, 16)


def _merge32(ak, av, bk, bv):
    rbk = lax.rev(bk, (0,))
    rbv = lax.rev(bv, (0,))
    c = ak <= rbk
    lok = jnp.where(c, ak, rbk)
    lov = jnp.where(c, av, rbv)
    hik = jnp.where(c, rbk, ak)
    hiv = jnp.where(c, rbv, av)
    lok, lov = plsc.sort_key_val(lok, lov)
    hik, hiv = plsc.sort_key_val(hik, hiv)
    return lok, lov, hik, hiv


def _base_pass(ksrc, kd, vd):
    def body(b, carry):
        off = b * 16
        kk = ksrc[pl.ds(off, 16)]
        vv = _iota16() + off
        sk, sv = plsc.sort_key_val(kk, vv)
        kd[pl.ds(off, 16)] = sk
        vd[pl.ds(off, 16)] = sv
        return carry

    lax.fori_loop(0, _NV, body, 0)


def _merge_pass(ks, vs, kd, vd, m):
    npairs = _N // (2 * m)
    nsteps = (2 * m) // 16 - 2

    def pair_body(pi, carry):
        a0 = pi * (2 * m)
        a_end = a0 + m
        b_end = a0 + 2 * m
        ak = ks[pl.ds(a0, 16)]
        av = vs[pl.ds(a0, 16)]
        bk = ks[pl.ds(a_end, 16)]
        bv = vs[pl.ds(a_end, 16)]
        lok, lov, hk, hv = _merge32(ak, av, bk, bv)
        kd[pl.ds(a0, 16)] = lok
        vd[pl.ds(a0, 16)] = lov
        ak = ks[pl.ds(a0 + 16, 16)]
        av = vs[pl.ds(a0 + 16, 16)]
        bk = ks[pl.ds(a_end + 16, 16)]
        bv = vs[pl.ds(a_end + 16, 16)]

        def step(t, c):
            apos, bpos, ak, av, bk, bv, hk, hv = c
            o = a0 + 16 * (t + 1)
            va = apos < a_end
            vb = bpos < b_end
            take_a = jnp.logical_and(
                va, jnp.logical_or(jnp.logical_not(vb), ak[0] <= bk[0])
            )
            ck = jnp.where(take_a, ak, bk)
            cv = jnp.where(take_a, av, bv)
            lk, lv, hk2, hv2 = _merge32(hk, hv, ck, cv)
            kd[pl.ds(o, 16)] = lk
            vd[pl.ds(o, 16)] = lv
            addr = jnp.where(take_a, apos, bpos) + 16
            fk = ks[pl.ds(addr, 16)]
            fv = vs[pl.ds(addr, 16)]
            ak2 = jnp.where(take_a, fk, ak)
            av2 = jnp.where(take_a, fv, av)
            bk2 = jnp.where(take_a, bk, fk)
            bv2 = jnp.where(take_a, bv, fv)
            apos2 = jnp.where(take_a, apos + 16, apos)
            bpos2 = jnp.where(take_a, bpos, bpos + 16)
            return apos2, bpos2, ak2, av2, bk2, bv2, hk2, hv2

        init = (a0 + 16, a_end + 16, ak, av, bk, bv, hk, hv)
        c = lax.fori_loop(0, nsteps, step, init)
        hk, hv = c[6], c[7]
        o_last = b_end - 16
        kd[pl.ds(o_last, 16)] = hk
        vd[pl.ds(o_last, 16)] = hv
        return carry

    lax.fori_loop(0, npairs, pair_body, 0)


def _rank_scatter(kf, vf, st, rdst):
    def fwd(b, carry):
        off = b * 16
        k = kf[pl.ds(off, 16)]
        pidx = _iota16() + off
        prevk = plsc.load_gather(kf, [jnp.maximum(pidx - 1, 0)])
        bnd = jnp.logical_or(k != prevk, pidx == 0)
        cand = jnp.where(bnd, pidx, 0)
        cm = jnp.maximum(plsc.cummax(cand), carry)
        st[pl.ds(off, 16)] = cm
        return lax.reduce_max(cm, (0,))

    lax.fori_loop(0, _NV, fwd, jnp.int32(0))

    def bwd(t, carry):
        b = _NV - 1 - t
        off = b * 16
        k = kf[pl.ds(off, 16)]
        pidx = _iota16() + off
        nxtk = plsc.load_gather(kf, [jnp.minimum(pidx + 1, _N - 1)])
        endb = jnp.logical_or(k != nxtk, pidx == _N - 1)
        cand = jnp.where(endb, pidx, _BIG)
        sfx = lax.rev(-plsc.cummax(-lax.rev(cand, (0,))), (0,))
        end = jnp.minimum(sfx, carry)
        s = st[pl.ds(off, 16)]
        rank_c = (s + end).astype(jnp.float32) * 0.5 + (1.0 - (_N + 1) / 2.0)
        v = vf[pl.ds(off, 16)]
        plsc.store_scatter(rdst, [v], rank_c)
        return lax.reduce_min(end, (0,))

    lax.fori_loop(0, _NV, bwd, jnp.int32(_N))


def kernel(pred_y, true_y):
    b, n = pred_y.shape
    mesh = plsc.VectorSubcoreMesh(core_axis_name="c", subcore_axis_name="s")
    nworkers = mesh.num_cores * mesh.num_subcores
    rows_per = b // nworkers

    def body(x_hbm, y_hbm, out_hbm, kA, kB, vA, vB, st, rx, ry, res):
        wid = lax.axis_index("s") * mesh.num_cores + lax.axis_index("c")

        def do_array(src_hbm, r, rdst):
            pltpu.sync_copy(src_hbm.at[r], kA.at[pl.ds(0, _N)])
            _base_pass(kA, kB, vB)
            _merge_pass(kB, vB, kA, vA, 16)
            _merge_pass(kA, vA, kB, vB, 32)
            _merge_pass(kB, vB, kA, vA, 64)
            _merge_pass(kA, vA, kB, vB, 128)
            _merge_pass(kB, vB, kA, vA, 256)
            _merge_pass(kA, vA, kB, vB, 512)
            _merge_pass(kB, vB, kA, vA, 1024)
            _merge_pass(kA, vA, kB, vB, 2048)
            _rank_scatter(kB, vB, st, rdst)

        def row_body(rloc, carry):
            r = wid * rows_per + rloc
            do_array(x_hbm, r, rx)
            do_array(y_hbm, r, ry)

            def dot_body(i, c):
                axy, axx, ayy = c
                off = i * 16
                a = rx[pl.ds(off, 16)]
                cc = ry[pl.ds(off, 16)]
                return axy + a * cc, axx + a * a, ayy + cc * cc

            z = jnp.zeros((16,), jnp.float32)
            axy, axx, ayy = lax.fori_loop(0, _NV, dot_body, (z, z, z))
            num = lax.reduce_sum(axy, (0,))
            den2 = lax.reduce_sum(axx, (0,)) * lax.reduce_sum(ayy, (0,))
            idx_n = jnp.full((16,), rloc, jnp.int32)
            idx_d = jnp.full((16,), rloc + 8, jnp.int32)
            lane0 = _iota16() == 0
            plsc.store_scatter(res, [idx_n], jnp.full((16,), num), mask=lane0)
            plsc.store_scatter(res, [idx_d], jnp.full((16,), den2), mask=lane0)
            return carry

        lax.fori_loop(0, rows_per, row_body, 0)
        pltpu.sync_copy(res, out_hbm.at[wid])

    k = pl.kernel(
        body,
        out_type=jax.ShapeDtypeStruct((nworkers, 16), jnp.float32),
        mesh=mesh,
        compiler_params=pltpu.CompilerParams(needs_layout_passes=False),
        scratch_types=[
            pltpu.VMEM((_N + 16,), jnp.float32),
            pltpu.VMEM((_N + 16,), jnp.float32),
            pltpu.VMEM((_N + 16,), jnp.int32),
            pltpu.VMEM((_N + 16,), jnp.int32),
            pltpu.VMEM((_N,), jnp.int32),
            pltpu.VMEM((_N,), jnp.float32),
            pltpu.VMEM((_N,), jnp.float32),
            pltpu.VMEM((16,), jnp.float32),
        ],
    )
    out = k(pred_y, true_y)
    num = out[:, 0:8].reshape(b)
    den2 = out[:, 8:16].reshape(b)
    return num / jnp.sqrt(den2 + _EPS)


# speedup vs baseline: 10.8549x; 1.1128x over previous
import jax
import jax.numpy as jnp
from jax import lax
from jax.experimental import pallas as pl
from jax.experimental.pallas import tpu as pltpu
from jax.experimental.pallas import tpu_sc as plsc

_N = 4096
_NV = _N // 16
_EPS = 1e-8
_BIG = _N
_MININT = -2147483648


def _iota16():
    return lax.iota(jnp.int32, 16)


def _build_keys(raw, kd):
    def body(b, c):
        off = b * 16
        x = raw[pl.ds(off, 16)]
        x = jnp.where(x == 0.0, 0.0, x)
        i = lax.bitcast_convert_type(x, jnp.int32)
        kd[pl.ds(off, 16)] = jnp.where(i < 0, ~i, i | jnp.int32(_MININT))
        return c

    lax.fori_loop(0, _NV, body, 0)


def _radix_pass(src_k, src_v, dst_k, dst_v, cnt, shift, twist_out, first):
    ones = jnp.ones((16,), jnp.int32)
    zeros = jnp.zeros((16,), jnp.int32)

    def zero(i, c):
        cnt[pl.ds(i * 16, 16)] = zeros
        return c

    lax.fori_loop(0, _NV, zero, 0)

    def digits(k):
        d = jnp.bitwise_and(lax.shift_right_logical(k, shift), 255)
        return (d << 4) + _iota16()

    def s1(b, c):
        idx = digits(src_k[pl.ds(b * 16, 16)])
        plsc.addupdate_scatter(cnt, [idx], ones)
        return c

    lax.fori_loop(0, _NV, s1, 0)

    def csum(dg, carry):
        c0 = cnt[pl.ds(dg * 16, 16)]
        incl = plsc.cumsum(c0)
        tot = lax.reduce_sum(c0, (0,))
        cnt[pl.ds(dg * 16, 16)] = incl - c0 + carry
        return carry + tot

    lax.fori_loop(0, _NV, csum, jnp.int32(0))

    def s2(b, c):
        off = b * 16
        k = src_k[pl.ds(off, 16)]
        idx = digits(k)
        pos = plsc.load_gather(cnt, [idx])
        v = _iota16() + off if first else src_v[pl.ds(off, 16)]
        if twist_out:
            w = (jnp.bitwise_and(pos, 255) << 4) | lax.shift_right_logical(
                pos, 8
            )
        else:
            w = pos
        plsc.store_scatter(dst_k, [w], k)
        plsc.store_scatter(dst_v, [w], v)
        plsc.addupdate_scatter(cnt, [idx], ones)
        return c

    lax.fori_loop(0, _NV, s2, 0)


def _rank_scatter(kf, vf, st, rdst):
    def fwd(b, carry):
        off = b * 16
        k = kf[pl.ds(off, 16)]
        pidx = _iota16() + off
        prevk = plsc.load_gather(kf, [jnp.maximum(pidx - 1, 0)])
        bnd = jnp.logical_or(k != prevk, pidx == 0)
        cand = jnp.where(bnd, pidx, 0)
        cm = jnp.maximum(plsc.cummax(cand), carry)
        st[pl.ds(off, 16)] = cm
        return lax.reduce_max(cm, (0,))

    lax.fori_loop(0, _NV, fwd, jnp.int32(0))

    def bwd(t, carry):
        b = _NV - 1 - t
        off = b * 16
        k = kf[pl.ds(off, 16)]
        pidx = _iota16() + off
        nxtk = plsc.load_gather(kf, [jnp.minimum(pidx + 1, _N - 1)])
        endb = jnp.logical_or(k != nxtk, pidx == _N - 1)
        cand = jnp.where(endb, pidx, _BIG)
        sfx = lax.rev(-plsc.cummax(-lax.rev(cand, (0,))), (0,))
        end = jnp.minimum(sfx, carry)
        s = st[pl.ds(off, 16)]
        rank_c = (s + end).astype(jnp.float32) * 0.5 + (1.0 - (_N + 1) / 2.0)
        v = vf[pl.ds(off, 16)]
        plsc.store_scatter(rdst, [v], rank_c)
        return lax.reduce_min(end, (0,))

    lax.fori_loop(0, _NV, bwd, jnp.int32(_N))


def kernel(pred_y, true_y):
    b, n = pred_y.shape
    mesh = plsc.VectorSubcoreMesh(core_axis_name="c", subcore_axis_name="s")
    nworkers = mesh.num_cores * mesh.num_subcores
    rows_per = b // nworkers

    def body(x_hbm, y_hbm, out_hbm, raw, kA, kB, vA, vB, cnt, st, rx, ry, res):
        wid = lax.axis_index("s") * mesh.num_cores + lax.axis_index("c")

        def do_array(src_hbm, r, rdst):
            pltpu.sync_copy(src_hbm.at[r], raw)
            _build_keys(raw, kA)
            _radix_pass(kA, vA, kB, vB, cnt, 0, True, True)
            _radix_pass(kB, vB, kA, vA, cnt, 8, True, False)
            _radix_pass(kA, vA, kB, vB, cnt, 16, True, False)
            _radix_pass(kB, vB, kA, vA, cnt, 24, False, False)
            _rank_scatter(kA, vA, st, rdst)

        def row_body(rloc, carry):
            r = wid * rows_per + rloc
            do_array(x_hbm, r, rx)
            do_array(y_hbm, r, ry)

            def dot_body(i, c):
                axy, axx, ayy = c
                off = i * 16
                a = rx[pl.ds(off, 16)]
                cc = ry[pl.ds(off, 16)]
                return axy + a * cc, axx + a * a, ayy + cc * cc

            z = jnp.zeros((16,), jnp.float32)
            axy, axx, ayy = lax.fori_loop(0, _NV, dot_body, (z, z, z))
            num = lax.reduce_sum(axy, (0,))
            den2 = lax.reduce_sum(axx, (0,)) * lax.reduce_sum(ayy, (0,))
            idx_n = jnp.full((16,), rloc, jnp.int32)
            idx_d = jnp.full((16,), rloc + 8, jnp.int32)
            lane0 = _iota16() == 0
            plsc.store_scatter(res, [idx_n], jnp.full((16,), num), mask=lane0)
            plsc.store_scatter(res, [idx_d], jnp.full((16,), den2), mask=lane0)
            return carry

        lax.fori_loop(0, rows_per, row_body, 0)
        pltpu.sync_copy(res, out_hbm.at[wid])

    k = pl.kernel(
        body,
        out_type=jax.ShapeDtypeStruct((nworkers, 16), jnp.float32),
        mesh=mesh,
        compiler_params=pltpu.CompilerParams(needs_layout_passes=False),
        scratch_types=[
            pltpu.VMEM((_N,), jnp.float32),
            pltpu.VMEM((_N,), jnp.int32),
            pltpu.VMEM((_N,), jnp.int32),
            pltpu.VMEM((_N,), jnp.int32),
            pltpu.VMEM((_N,), jnp.int32),
            pltpu.VMEM((_N,), jnp.int32),
            pltpu.VMEM((_N,), jnp.int32),
            pltpu.VMEM((_N,), jnp.float32),
            pltpu.VMEM((_N,), jnp.float32),
            pltpu.VMEM((16,), jnp.float32),
        ],
    )
    out = k(pred_y, true_y)
    num = out[:, 0:8].reshape(b)
    den2 = out[:, 8:16].reshape(b)
    return num / jnp.sqrt(den2 + _EPS)


# speedup vs baseline: 14.5485x; 1.3403x over previous
import jax
import jax.numpy as jnp
from jax import lax
from jax.experimental import pallas as pl
from jax.experimental.pallas import tpu as pltpu
from jax.experimental.pallas import tpu_sc as plsc

_N = 4096
_NV = _N // 16
_EPS = 1e-8
_BIG = _N
_MININT = -2147483648
_UNROLL = 8


def _iota16():
    return lax.iota(jnp.int32, 16)


def _keys_from_raw(x):
    x = jnp.where(x == 0.0, 0.0, x)
    i = lax.bitcast_convert_type(x, jnp.int32)
    return jnp.where(i < 0, ~i, i | jnp.int32(_MININT))


def _radix_pass(src_k, src_v, dst_k, dst_v, cnt, shift, twist_out, first,
                raw=None):
    ones = jnp.ones((16,), jnp.int32)

    def digits(k):
        d = jnp.bitwise_and(lax.shift_right_logical(k, shift), 255)
        return (d << 4) + _iota16()

    def s1(b, c):
        off = b * 16
        if raw is not None:
            k = _keys_from_raw(raw[pl.ds(off, 16)])
            src_k[pl.ds(off, 16)] = k
        else:
            k = src_k[pl.ds(off, 16)]
        plsc.addupdate_scatter(cnt, [digits(k)], ones)
        return c

    lax.fori_loop(0, _NV, s1, 0, unroll=_UNROLL)

    def csum(dg, carry):
        c0 = cnt[pl.ds(dg * 16, 16)]
        incl = plsc.cumsum(c0)
        tot = lax.reduce_sum(c0, (0,))
        cnt[pl.ds(dg * 16, 16)] = incl - c0 + carry
        return carry + tot

    lax.fori_loop(0, _NV, csum, jnp.int32(0), unroll=_UNROLL)

    def s2(b, c):
        off = b * 16
        k = src_k[pl.ds(off, 16)]
        idx = digits(k)
        pos = plsc.load_gather(cnt, [idx])
        v = _iota16() + off if first else src_v[pl.ds(off, 16)]
        if twist_out:
            w = (jnp.bitwise_and(pos, 255) << 4) | lax.shift_right_logical(
                pos, 8
            )
        else:
            w = pos
        plsc.store_scatter(dst_k, [w], k)
        plsc.store_scatter(dst_v, [w], v)
        plsc.addupdate_scatter(cnt, [idx], ones)
        return c

    lax.fori_loop(0, _NV, s2, 0, unroll=_UNROLL)


def _rank_scatter(kf, vf, st, rdst):
    def fwd(b, carry):
        off = b * 16
        k = kf[pl.ds(off, 16)]
        pidx = _iota16() + off
        prevk = plsc.load_gather(kf, [jnp.maximum(pidx - 1, 0)])
        bnd = jnp.logical_or(k != prevk, pidx == 0)
        cand = jnp.where(bnd, pidx, 0)
        cm = jnp.maximum(plsc.cummax(cand), carry)
        st[pl.ds(off, 16)] = cm
        return lax.reduce_max(cm, (0,))

    lax.fori_loop(0, _NV, fwd, jnp.int32(0), unroll=_UNROLL)

    def bwd(t, carry):
        b = _NV - 1 - t
        off = b * 16
        k = kf[pl.ds(off, 16)]
        pidx = _iota16() + off
        nxtk = plsc.load_gather(kf, [jnp.minimum(pidx + 1, _N - 1)])
        endb = jnp.logical_or(k != nxtk, pidx == _N - 1)
        cand = jnp.where(endb, pidx, _BIG)
        sfx = lax.rev(-plsc.cummax(-lax.rev(cand, (0,))), (0,))
        end = jnp.minimum(sfx, carry)
        s = st[pl.ds(off, 16)]
        rank_c = (s + end).astype(jnp.float32) * 0.5 + (1.0 - (_N + 1) / 2.0)
        v = vf[pl.ds(off, 16)]
        plsc.store_scatter(rdst, [v], rank_c)
        return lax.reduce_min(end, (0,))

    lax.fori_loop(0, _NV, bwd, jnp.int32(_N), unroll=_UNROLL)


def kernel(pred_y, true_y):
    b, n = pred_y.shape
    mesh = plsc.VectorSubcoreMesh(core_axis_name="c", subcore_axis_name="s")
    nworkers = mesh.num_cores * mesh.num_subcores
    rows_per = b // nworkers

    def body(x_hbm, y_hbm, out_hbm, raw, kA, kB, vA, vB,
             c0, c1, c2, c3, st, rx, ry, res):
        wid = lax.axis_index("s") * mesh.num_cores + lax.axis_index("c")
        zeros = jnp.zeros((16,), jnp.int32)

        def do_array(src_hbm, r, rdst):
            pltpu.sync_copy(src_hbm.at[r], raw)

            def zero(i, c):
                c0[pl.ds(i * 16, 16)] = zeros
                c1[pl.ds(i * 16, 16)] = zeros
                c2[pl.ds(i * 16, 16)] = zeros
                c3[pl.ds(i * 16, 16)] = zeros
                return c

            lax.fori_loop(0, _NV, zero, 0, unroll=_UNROLL)
            _radix_pass(kA, vA, kB, vB, c0, 0, True, True, raw=raw)
            _radix_pass(kB, vB, kA, vA, c1, 8, True, False)
            _radix_pass(kA, vA, kB, vB, c2, 16, True, False)
            _radix_pass(kB, vB, kA, vA, c3, 24, False, False)
            _rank_scatter(kA, vA, st, rdst)

        def row_body(rloc, carry):
            r = wid * rows_per + rloc
            do_array(x_hbm, r, rx)
            do_array(y_hbm, r, ry)

            def dot_body(i, c):
                axy, axx, ayy = c
                off = i * 16
                a = rx[pl.ds(off, 16)]
                cc = ry[pl.ds(off, 16)]
                return axy + a * cc, axx + a * a, ayy + cc * cc

            z = jnp.zeros((16,), jnp.float32)
            axy, axx, ayy = lax.fori_loop(
                0, _NV, dot_body, (z, z, z), unroll=_UNROLL
            )
            num = lax.reduce_sum(axy, (0,))
            den2 = lax.reduce_sum(axx, (0,)) * lax.reduce_sum(ayy, (0,))
            idx_n = jnp.full((16,), rloc, jnp.int32)
            idx_d = jnp.full((16,), rloc + 8, jnp.int32)
            lane0 = _iota16() == 0
            plsc.store_scatter(res, [idx_n], jnp.full((16,), num), mask=lane0)
            plsc.store_scatter(res, [idx_d], jnp.full((16,), den2), mask=lane0)
            return carry

        lax.fori_loop(0, rows_per, row_body, 0)
        pltpu.sync_copy(res, out_hbm.at[wid])

    k = pl.kernel(
        body,
        out_type=jax.ShapeDtypeStruct((nworkers, 16), jnp.float32),
        mesh=mesh,
        compiler_params=pltpu.CompilerParams(needs_layout_passes=False),
        scratch_types=[
            pltpu.VMEM((_N,), jnp.float32),
            pltpu.VMEM((_N,), jnp.int32),
            pltpu.VMEM((_N,), jnp.int32),
            pltpu.VMEM((_N,), jnp.int32),
            pltpu.VMEM((_N,), jnp.int32),
            pltpu.VMEM((_N,), jnp.int32),
            pltpu.VMEM((_N,), jnp.int32),
            pltpu.VMEM((_N,), jnp.int32),
            pltpu.VMEM((_N,), jnp.int32),
            pltpu.VMEM((_N,), jnp.int32),
            pltpu.VMEM((_N,), jnp.float32),
            pltpu.VMEM((_N,), jnp.float32),
            pltpu.VMEM((16,), jnp.float32),
        ],
    )
    out = k(pred_y, true_y)
    num = out[:, 0:8].reshape(b)
    den2 = out[:, 8:16].reshape(b)
    return num / jnp.sqrt(den2 + _EPS)
